# baseline (speedup 1.0000x reference)
import functools

import jax
import jax.numpy as jnp
from jax import lax
from jax.experimental import pallas as pl
from jax.experimental.pallas import tpu as pltpu
from jax.experimental.pallas import tpu_sc as plsc

S_EMB = 128
D = 512
AA_H = 152
AA_L = 139
T = AA_H + AA_L
HALF = 2048
REP = 8

_NC = 2
_NS = 16
_PER_W = HALF // (_NC * _NS)
_DMAS_PER_W = _PER_W // REP


def _mlp_block_body(emb_ref, w1_ref, b1_ref, g_ref, bln_ref, w2_ref, b2_ref,
                    out_ref):
    e = emb_ref[...]
    h = lax.dot_general(e, w1_ref[...], (((1,), (1,)), ((), ())),
                        preferred_element_type=jnp.float32)
    h = h + b1_ref[...]
    mu = jnp.mean(h, axis=-1, keepdims=True)
    var = jnp.mean((h - mu) ** 2, axis=-1, keepdims=True)
    h = (h - mu) / jnp.sqrt(var + 1e-5) * g_ref[...] + bln_ref[...]
    h = jnp.maximum(h, 0.0)
    h = lax.dot_general(h, w2_ref[...], (((1,), (1,)), ((), ())),
                        preferred_element_type=jnp.float32) + b2_ref[...]
    t = lax.broadcasted_iota(jnp.int32, (T, 1), 0)
    period = jnp.where(t < AA_H, h[0:1, :], h[1:2, :])
    out_ref[...] = jnp.broadcast_to(period[None], (REP, T, D))


def _sc_fanout_body(block_hbm, out_hbm, spmem):
    cid = lax.axis_index("c")
    sid = lax.axis_index("s")

    @pl.when(sid == 0)
    def _stage():
        pltpu.sync_copy(block_hbm, spmem)

    plsc.subcore_barrier()
    base = (cid * _NS + sid) * _PER_W
    for k in range(_DMAS_PER_W):
        pltpu.sync_copy(spmem, out_hbm.at[pl.ds(base + k * REP, REP)])


def kernel(side, emb_table, W1, b1, ln_g, ln_b, W2, b2):
    del side
    block = pl.pallas_call(
        _mlp_block_body,
        out_shape=jax.ShapeDtypeStruct((REP, T, D), jnp.float32),
    )(emb_table, W1, b1.reshape(1, D), ln_g.reshape(1, D),
      ln_b.reshape(1, D), W2, b2.reshape(1, D))

    mesh = plsc.VectorSubcoreMesh(core_axis_name="c", subcore_axis_name="s")
    fanout = functools.partial(
        pl.kernel,
        mesh=mesh,
        out_type=jax.ShapeDtypeStruct((HALF, T, D), jnp.float32),
        scratch_types=[pltpu.VMEM_SHARED((REP, T, D), jnp.float32)],
    )(_sc_fanout_body)
    return fanout(block)


# speedup vs baseline: 1.0058x; 1.0058x over previous
import functools

import jax
import jax.numpy as jnp
from jax import lax
from jax.experimental import pallas as pl
from jax.experimental.pallas import tpu as pltpu
from jax.experimental.pallas import tpu_sc as plsc

S_EMB = 128
D = 512
AA_H = 152
AA_L = 139
T = AA_H + AA_L
HALF = 2048
REP = 8

_NC = 2
_NS = 16
_PER_W = HALF // (_NC * _NS)
_DMAS_PER_W = _PER_W // REP


def _mlp_block_body(emb_ref, w1_ref, b1_ref, g_ref, bln_ref, w2_ref, b2_ref,
                    out_ref):
    e = emb_ref[...]
    h = lax.dot_general(e, w1_ref[...], (((1,), (1,)), ((), ())),
                        preferred_element_type=jnp.float32)
    h = h + b1_ref[...]
    mu = jnp.mean(h, axis=-1, keepdims=True)
    var = jnp.mean((h - mu) ** 2, axis=-1, keepdims=True)
    h = (h - mu) / jnp.sqrt(var + 1e-5) * g_ref[...] + bln_ref[...]
    h = jnp.maximum(h, 0.0)
    h = lax.dot_general(h, w2_ref[...], (((1,), (1,)), ((), ())),
                        preferred_element_type=jnp.float32) + b2_ref[...]
    t = lax.broadcasted_iota(jnp.int32, (T, 1), 0)
    period = jnp.where(t < AA_H, h[0:1, :], h[1:2, :])
    out_ref[...] = jnp.broadcast_to(period[None], (REP, T, D))


def _sc_fanout_body(block_hbm, out_hbm, spmem, sem):
    cid = lax.axis_index("c")
    sid = lax.axis_index("s")

    @pl.when(sid == 0)
    def _stage():
        pltpu.sync_copy(block_hbm, spmem)

    plsc.subcore_barrier()
    base = (cid * _NS + sid) * _PER_W
    copies = [
        pltpu.make_async_copy(
            spmem, out_hbm.at[pl.ds(base + k * REP, REP)], sem)
        for k in range(_DMAS_PER_W)
    ]
    for c in copies:
        c.start()
    for c in copies:
        c.wait()


def kernel(side, emb_table, W1, b1, ln_g, ln_b, W2, b2):
    del side
    block = pl.pallas_call(
        _mlp_block_body,
        out_shape=jax.ShapeDtypeStruct((REP, T, D), jnp.float32),
    )(emb_table, W1, b1.reshape(1, D), ln_g.reshape(1, D),
      ln_b.reshape(1, D), W2, b2.reshape(1, D))

    mesh = plsc.VectorSubcoreMesh(core_axis_name="c", subcore_axis_name="s")
    fanout = functools.partial(
        pl.kernel,
        mesh=mesh,
        out_type=jax.ShapeDtypeStruct((HALF, T, D), jnp.float32),
        scratch_types=[pltpu.VMEM_SHARED((REP, T, D), jnp.float32),
                       pltpu.SemaphoreType.DMA],
    )(_sc_fanout_body)
    return fanout(block)
